# baseline (speedup 1.0000x reference)
import jax
import jax.numpy as jnp
from jax import lax
from jax.experimental import pallas as pl
from jax.experimental.pallas import tpu as pltpu

B = 4096
C8 = 8
NK = 256
DOUT = 1000
GP = 32
BLK = 512
NBLK = B // BLK

_TLHS = (((0,), (0,)), ((), ()))


def _body(xt_ref, w_ref, t_ref, h_ref, lut_ref, out_ref, lut_bf):
    @pl.when(pl.program_id(0) == 0)
    def _():
        for c in range(C8):
            lut_bf[c] = lut_ref[c].astype(jnp.bfloat16)

    g = jnp.dot(w_ref[...], xt_ref[...], preferred_element_type=jnp.float32)
    s = jnp.sign(g - t_ref[...] - 0.0001)
    ids = lax.broadcasted_iota(jnp.int32, (NK, BLK), 0)
    for c in range(C8):
        sc = lax.dot_general(h_ref[...], s[c * GP:c * GP + 30, :], _TLHS,
                             preferred_element_type=jnp.float32)
        m = jnp.max(sc, axis=0, keepdims=True)
        idx = jnp.min(jnp.where(sc == m, ids, NK), axis=0, keepdims=True)
        oh = (ids == idx).astype(jnp.bfloat16)
        out_ref[c] = jnp.dot(lut_bf[c], oh, preferred_element_type=jnp.float32)


def kernel(x, S, H, T, LUT):
    f32 = jnp.float32
    S2 = S.reshape(C8, 2, 2, 15)
    E = jnp.eye(32, dtype=f32).reshape(C8, 2, 2, 32)
    W4 = jnp.einsum('cjdk,cjdm->cjkm', S2, E).reshape(C8, 30, 32)
    W4 = jnp.pad(W4, ((0, 0), (0, 2), (0, 0))).reshape(C8 * GP, 32)
    T4 = jnp.pad(T.reshape(C8, 30), ((0, 0), (0, 2))).reshape(C8 * GP, 1)
    lut_t = LUT.transpose(0, 2, 1)
    out_t = pl.pallas_call(
        _body,
        grid=(NBLK,),
        in_specs=[
            pl.BlockSpec((32, BLK), lambda i: (0, i)),
            pl.BlockSpec((C8 * GP, 32), lambda i: (0, 0)),
            pl.BlockSpec((C8 * GP, 1), lambda i: (0, 0)),
            pl.BlockSpec((30, NK), lambda i: (0, 0)),
            pl.BlockSpec((C8, DOUT, NK), lambda i: (0, 0, 0)),
        ],
        out_specs=pl.BlockSpec((C8, DOUT, BLK), lambda i: (0, 0, i)),
        out_shape=jax.ShapeDtypeStruct((C8, DOUT, B), f32),
        scratch_shapes=[pltpu.VMEM((C8, DOUT, NK), jnp.bfloat16)],
    )(x.T, W4, T4, H, lut_t)
    return out_t.transpose(2, 0, 1)


# speedup vs baseline: 1.0107x; 1.0107x over previous
import jax
import jax.numpy as jnp
from jax import lax
from jax.experimental import pallas as pl
from jax.experimental.pallas import tpu as pltpu

B = 4096
C8 = 8
NK = 256
DOUT = 1000
GP = 32
BLK = 256
NBLK = B // BLK

_TLHS = (((0,), (0,)), ((), ()))


def _body(xt_ref, w_ref, t_ref, h_ref, lut_ref, out_ref, lut_bf):
    @pl.when(pl.program_id(0) == 0)
    def _():
        for c in range(C8):
            lut_bf[c] = lut_ref[c].astype(jnp.bfloat16)

    g = jnp.dot(w_ref[...], xt_ref[...], preferred_element_type=jnp.float32)
    s = jnp.sign(g - t_ref[...] - 0.0001)
    ids = lax.broadcasted_iota(jnp.int32, (NK, BLK), 0)
    for c in range(C8):
        sc = lax.dot_general(h_ref[...], s[c * GP:c * GP + 30, :], _TLHS,
                             preferred_element_type=jnp.float32)
        m = jnp.max(sc, axis=0, keepdims=True)
        idx = jnp.min(jnp.where(sc == m, ids, NK), axis=0, keepdims=True)
        oh = (ids == idx).astype(jnp.bfloat16)
        out_ref[c] = jnp.dot(lut_bf[c], oh, preferred_element_type=jnp.float32)


def kernel(x, S, H, T, LUT):
    f32 = jnp.float32
    S2 = S.reshape(C8, 2, 2, 15)
    E = jnp.eye(32, dtype=f32).reshape(C8, 2, 2, 32)
    W4 = jnp.einsum('cjdk,cjdm->cjkm', S2, E).reshape(C8, 30, 32)
    W4 = jnp.pad(W4, ((0, 0), (0, 2), (0, 0))).reshape(C8 * GP, 32)
    T4 = jnp.pad(T.reshape(C8, 30), ((0, 0), (0, 2))).reshape(C8 * GP, 1)
    lut_t = LUT.transpose(0, 2, 1)
    out_t = pl.pallas_call(
        _body,
        grid=(NBLK,),
        in_specs=[
            pl.BlockSpec((32, BLK), lambda i: (0, i)),
            pl.BlockSpec((C8 * GP, 32), lambda i: (0, 0)),
            pl.BlockSpec((C8 * GP, 1), lambda i: (0, 0)),
            pl.BlockSpec((30, NK), lambda i: (0, 0)),
            pl.BlockSpec((C8, DOUT, NK), lambda i: (0, 0, 0)),
        ],
        out_specs=pl.BlockSpec((C8, DOUT, BLK), lambda i: (0, 0, i)),
        out_shape=jax.ShapeDtypeStruct((C8, DOUT, B), f32),
        scratch_shapes=[pltpu.VMEM((C8, DOUT, NK), jnp.bfloat16)],
    )(x.T, W4, T4, H, lut_t)
    return out_t.transpose(2, 0, 1)
